# baseline (speedup 1.0000x reference)
import functools

import jax
import jax.numpy as jnp
from jax import lax
from jax.experimental import pallas as pl
from jax.experimental.pallas import tpu as pltpu
from jax.experimental.pallas import tpu_sc as plsc

_M = 16384
_D = 64
_KP = 8192
_KS = 1024
_B = 16
_N = 1024
_CC = 0.25
_RT = 512
_KT = 512
_STRIP = 4096
_NW = 32


def _vq1_body(x_ref, w_ref, idx_ref, mind_ref):
    x = x_ref[...]
    q = x * x
    g = q[:, 0:8]
    for t in range(1, 8):
        g = g + q[:, 8 * t:8 * t + 8]
    h2 = g[:, 0:4] + g[:, 4:8]
    i2 = h2[:, 0:2] + h2[:, 2:4]
    x2 = i2[:, 0:1] + i2[:, 1:2]
    xb = (2.0 * x).astype(jnp.bfloat16).astype(jnp.float32)
    ones_row = jnp.ones((1, _D), jnp.float32)
    iota_k = lax.broadcasted_iota(jnp.int32, (_RT, _KT), 1)

    def sub_step(j, carry):
        sv, si = carry
        wk = w_ref[pl.ds(j * _KT, _KT), :]
        xw2 = lax.dot_general(xb, wk, (((1,), (1,)), ((), ())),
                              preferred_element_type=jnp.float32)
        w2 = lax.dot_general(ones_row, wk * wk, (((1,), (1,)), ((), ())),
                             preferred_element_type=jnp.float32)
        d = (x2 + w2) - xw2
        lm = jnp.min(d, axis=1, keepdims=True)
        li = jnp.min(jnp.where(d == lm, iota_k, _KT), axis=1, keepdims=True)
        pred = lm < sv
        return (jnp.where(pred, lm, sv),
                jnp.where(pred, li + j * _KT, si))

    def strip_step(c, carry):
        bv, bi, be = carry
        sv0 = jnp.full((_RT, 1), jnp.inf, jnp.float32)
        si0 = jnp.zeros((_RT, 1), jnp.int32)
        n_sub = _STRIP // _KT
        sv, si = lax.fori_loop(c * n_sub, (c + 1) * n_sub, sub_step,
                               (sv0, si0))
        pred = sv < bv
        bv = jnp.where(pred, sv, bv).astype(jnp.bfloat16).astype(jnp.float32)
        return (bv, jnp.where(pred, si, bi), jnp.where(pred, sv, be))

    bv0 = jnp.full((_RT, 1), jnp.inf, jnp.float32)
    bi0 = jnp.zeros((_RT, 1), jnp.int32)
    be0 = jnp.full((_RT, 1), jnp.inf, jnp.float32)
    _, bi, be = lax.fori_loop(0, _KP // _STRIP, strip_step, (bv0, bi0, be0))
    idx_ref[...] = bi
    mind_ref[...] = be


def _head_body(hist_ref, mind_ref, w1_ref, b1_ref, g1_ref, bb1_ref,
               w2_ref, b2_ref, g2_ref, bb2_ref, sw_ref,
               sq_ref, loss_ref, sidx_ref):
    hist = hist_ref[...]
    pd = (hist[:, :_KP] + hist[:, _KP:]) * (1.0 / _N)
    h = lax.dot_general(pd, w1_ref[...], (((1,), (0,)), ((), ())),
                        preferred_element_type=jnp.float32) + b1_ref[...]

    def ln(v, g, b):
        mu = jnp.mean(v, axis=-1, keepdims=True)
        var = jnp.mean((v - mu) ** 2, axis=-1, keepdims=True)
        return (v - mu) / jnp.sqrt(var + 1e-5) * g + b

    h = ln(h, g1_ref[...], bb1_ref[...])
    h = jnp.maximum(h, 0.0)
    h = lax.dot_general(h, w2_ref[...], (((1,), (0,)), ((), ())),
                        preferred_element_type=jnp.float32) + b2_ref[...]
    s = ln(h, g2_ref[...], bb2_ref[...])

    sw = sw_ref[...]
    s2 = jnp.sum(s * s, axis=1, keepdims=True)
    ones_row = jnp.ones((1, _D), jnp.float32)
    sw2 = lax.dot_general(ones_row, sw * sw, (((1,), (1,)), ((), ())),
                          preferred_element_type=jnp.float32)
    ssw = lax.dot_general(s, sw, (((1,), (1,)), ((), ())),
                          preferred_element_type=jnp.float32)
    d2 = (s2 + sw2) - 2.0 * ssw
    m2 = jnp.min(d2, axis=1, keepdims=True)
    iota_s = lax.broadcasted_iota(jnp.int32, (_B, _KS), 1)
    si = jnp.min(jnp.where(d2 == m2, iota_s, _KS), axis=1, keepdims=True)
    oh = (iota_s == si).astype(jnp.float32)
    q2 = lax.dot_general(oh, sw, (((1,), (0,)), ((), ())),
                         preferred_element_type=jnp.float32)

    diff = q2 - s
    msq = jnp.sum(diff * diff) * (1.0 / (_B * _D))
    state_loss = msq + _CC * msq
    pm = jnp.sum(mind_ref[...]) * (1.0 / (_M * _D))
    patch_loss = pm + _CC * pm
    total = patch_loss + state_loss

    sq_ref[...] = s + (q2 - s)
    loss_ref[...] = jnp.reshape(total, (1, 1))
    sidx_ref[...] = si


def _sc_body(x_hbm, w_hbm, idx_hbm, outq_hbm, hist_hbm,
             idx_v, q_v, x_v, hist_v, sem):
    wid = lax.axis_index("s") * 2 + lax.axis_index("c")
    base = wid * _RT
    pltpu.sync_copy(idx_hbm.at[wid], idx_v)
    for j in range(4):
        pltpu.async_copy(w_hbm.at[idx_v.at[j]],
                         q_v.at[pl.ds(j * 128, 128)], sem).wait()
    pltpu.sync_copy(x_hbm.at[pl.ds(base, _RT)], x_v)

    def rowfn(r, carry):
        for c in range(_D // 16):
            sl = pl.ds(c * 16, 16)
            xx = x_v[r, sl]
            q_v[r, sl] = xx + (q_v[r, sl] - xx)
        return carry

    lax.fori_loop(0, _RT, rowfn, 0)

    zero16 = jnp.zeros((16,), jnp.float32)

    def zfn(i, carry):
        hist_v[pl.ds(i * 16, 16)] = zero16
        return carry

    lax.fori_loop(0, _KP // 16, zfn, 0)

    ones16 = jnp.ones((16,), jnp.float32)
    lanes = lax.iota(jnp.int32, 16)

    def hfn(t, carry):
        ivec = idx_v[t // 8, pl.ds((t % 8) * 16, 16)]
        for l in range(16):
            plsc.addupdate_scatter(hist_v, [ivec], ones16,
                                   mask=lanes == l)
        return carry

    lax.fori_loop(0, _NW, hfn, 0)

    pltpu.sync_copy(q_v, outq_hbm.at[pl.ds(base, _RT)])
    pltpu.sync_copy(hist_v, hist_hbm.at[wid])


@functools.cache
def _sc_gather_hist():
    mesh = plsc.VectorSubcoreMesh(core_axis_name="c", subcore_axis_name="s")
    return pl.kernel(
        _sc_body,
        mesh=mesh,
        out_type=(jax.ShapeDtypeStruct((_M, _D), jnp.float32),
                  jax.ShapeDtypeStruct((_NW, _KP), jnp.float32)),
        scratch_types=(pltpu.VMEM((4, 128), jnp.int32),
                       pltpu.VMEM((_RT, _D), jnp.float32),
                       pltpu.VMEM((_RT, _D), jnp.float32),
                       pltpu.VMEM((_KP,), jnp.float32),
                       pltpu.SemaphoreType.DMA),
        compiler_params=pltpu.CompilerParams(needs_layout_passes=False,
                                             use_tc_tiling_on_sc=False),
    )


def _vq1_call(flat, w):
    return pl.pallas_call(
        _vq1_body,
        grid=(_M // _RT,),
        in_specs=[pl.BlockSpec((_RT, _D), lambda i: (i, 0)),
                  pl.BlockSpec((_KP, _D), lambda i: (0, 0))],
        out_specs=[pl.BlockSpec((_RT, 1), lambda i: (i, 0)),
                   pl.BlockSpec((_RT, 1), lambda i: (i, 0))],
        out_shape=[jax.ShapeDtypeStruct((_M, 1), jnp.int32),
                   jax.ShapeDtypeStruct((_M, 1), jnp.float32)],
    )(flat, w)


def _head_call(hist16, mind, w1, b1, g1, bb1, w2, b2, g2, bb2, sw):
    return pl.pallas_call(
        _head_body,
        out_shape=[jax.ShapeDtypeStruct((_B, _D), jnp.float32),
                   jax.ShapeDtypeStruct((1, 1), jnp.float32),
                   jax.ShapeDtypeStruct((_B, 1), jnp.int32)],
    )(hist16, mind, w1, b1, g1, bb1, w2, b2, g2, bb2, sw)


def kernel(patch_embeddings, patch_embed_w, state_embed_w, W1, b1,
           ln1_g, ln1_b, W2, b2, ln2_g, ln2_b):
    flat = patch_embeddings.reshape(_M, _D)
    idx_col, mind_col = _vq1_call(flat, patch_embed_w)
    patch_indices = idx_col.reshape(_B, _N)
    idx_sc = idx_col.reshape(_NW, 4, 128)
    patch_q_flat, hist = _sc_gather_hist()(flat, patch_embed_w, idx_sc)
    patch_quantized = patch_q_flat.reshape(_B, _N, _D)
    sq, loss11, sidx = _head_call(
        hist.reshape(_B, 2 * _KP), mind_col.reshape(128, 128),
        W1, b1.reshape(1, _D), ln1_g.reshape(1, _D), ln1_b.reshape(1, _D),
        W2, b2.reshape(1, _D), ln2_g.reshape(1, _D), ln2_b.reshape(1, _D),
        state_embed_w)
    return (patch_quantized, sq, loss11.reshape(()), patch_indices,
            sidx.reshape(_B))


# speedup vs baseline: 1.1848x; 1.1848x over previous
import functools

import jax
import jax.numpy as jnp
from jax import lax
from jax.experimental import pallas as pl
from jax.experimental.pallas import tpu as pltpu
from jax.experimental.pallas import tpu_sc as plsc

_M = 16384
_D = 64
_KP = 8192
_KS = 1024
_B = 16
_N = 1024
_CC = 0.25
_RT = 512
_KT = 1024
_STRIP = 4096
_NW = 32


def _vq1_body(x_ref, w_ref, idx_ref, mind_ref):
    x = x_ref[...]
    q = x * x
    g = q[:, 0:8]
    for t in range(1, 8):
        g = g + q[:, 8 * t:8 * t + 8]
    h2 = g[:, 0:4] + g[:, 4:8]
    i2 = h2[:, 0:2] + h2[:, 2:4]
    x2 = i2[:, 0:1] + i2[:, 1:2]
    xb = (2.0 * x).astype(jnp.bfloat16).astype(jnp.float32)
    ones_row = jnp.ones((1, _D), jnp.float32)
    iota_k = lax.broadcasted_iota(jnp.int32, (_RT, _KT), 1)

    def sub_step(j, carry):
        sv, si = carry
        wk = w_ref[pl.ds(j * _KT, _KT), :]
        xw2 = lax.dot_general(xb, wk, (((1,), (1,)), ((), ())),
                              preferred_element_type=jnp.float32)
        w2 = lax.dot_general(ones_row, wk * wk, (((1,), (1,)), ((), ())),
                             preferred_element_type=jnp.float32)
        d = (x2 + w2) - xw2
        lm = jnp.min(d, axis=1, keepdims=True)
        li = jnp.argmin(d, axis=1, keepdims=True).astype(jnp.int32)
        pred = lm < sv
        return (jnp.where(pred, lm, sv),
                jnp.where(pred, li + j * _KT, si))

    def strip_step(c, carry):
        bv, bi, be = carry
        sv0 = jnp.full((_RT, 1), jnp.inf, jnp.float32)
        si0 = jnp.zeros((_RT, 1), jnp.int32)
        n_sub = _STRIP // _KT
        sv, si = lax.fori_loop(c * n_sub, (c + 1) * n_sub, sub_step,
                               (sv0, si0))
        pred = sv < bv
        bv = jnp.where(pred, sv, bv).astype(jnp.bfloat16).astype(jnp.float32)
        return (bv, jnp.where(pred, si, bi), jnp.where(pred, sv, be))

    bv0 = jnp.full((_RT, 1), jnp.inf, jnp.float32)
    bi0 = jnp.zeros((_RT, 1), jnp.int32)
    be0 = jnp.full((_RT, 1), jnp.inf, jnp.float32)
    _, bi, be = lax.fori_loop(0, _KP // _STRIP, strip_step, (bv0, bi0, be0))
    idx_ref[...] = bi
    mind_ref[...] = be


def _head_body(hist_ref, mind_ref, w1_ref, b1_ref, g1_ref, bb1_ref,
               w2_ref, b2_ref, g2_ref, bb2_ref, sw_ref,
               sq_ref, loss_ref, sidx_ref):
    hist = hist_ref[...]
    pd = (hist[:, :_KP] + hist[:, _KP:]) * (1.0 / _N)
    h = lax.dot_general(pd, w1_ref[...], (((1,), (0,)), ((), ())),
                        preferred_element_type=jnp.float32) + b1_ref[...]

    def ln(v, g, b):
        mu = jnp.mean(v, axis=-1, keepdims=True)
        var = jnp.mean((v - mu) ** 2, axis=-1, keepdims=True)
        return (v - mu) / jnp.sqrt(var + 1e-5) * g + b

    h = ln(h, g1_ref[...], bb1_ref[...])
    h = jnp.maximum(h, 0.0)
    h = lax.dot_general(h, w2_ref[...], (((1,), (0,)), ((), ())),
                        preferred_element_type=jnp.float32) + b2_ref[...]
    s = ln(h, g2_ref[...], bb2_ref[...])

    sw = sw_ref[...]
    s2 = jnp.sum(s * s, axis=1, keepdims=True)
    ones_row = jnp.ones((1, _D), jnp.float32)
    sw2 = lax.dot_general(ones_row, sw * sw, (((1,), (1,)), ((), ())),
                          preferred_element_type=jnp.float32)
    ssw = lax.dot_general(s, sw, (((1,), (1,)), ((), ())),
                          preferred_element_type=jnp.float32)
    d2 = (s2 + sw2) - 2.0 * ssw
    m2 = jnp.min(d2, axis=1, keepdims=True)
    iota_s = lax.broadcasted_iota(jnp.int32, (_B, _KS), 1)
    si = jnp.min(jnp.where(d2 == m2, iota_s, _KS), axis=1, keepdims=True)
    oh = (iota_s == si).astype(jnp.float32)
    q2 = lax.dot_general(oh, sw, (((1,), (0,)), ((), ())),
                         preferred_element_type=jnp.float32)

    diff = q2 - s
    msq = jnp.sum(diff * diff) * (1.0 / (_B * _D))
    state_loss = msq + _CC * msq
    pm = jnp.sum(mind_ref[...]) * (1.0 / (_M * _D))
    patch_loss = pm + _CC * pm
    total = patch_loss + state_loss

    sq_ref[...] = s + (q2 - s)
    loss_ref[...] = jnp.reshape(total, (1, 1))
    sidx_ref[...] = si


def _sc_body(x_hbm, w_hbm, idx_hbm, outq_hbm, hist_hbm,
             idx_v, q_v, x_v, hist_v, sem):
    wid = lax.axis_index("s") * 2 + lax.axis_index("c")
    base = wid * _RT
    pltpu.sync_copy(idx_hbm.at[wid], idx_v)
    for j in range(4):
        pltpu.async_copy(w_hbm.at[idx_v.at[j]],
                         q_v.at[pl.ds(j * 128, 128)], sem).wait()
    pltpu.sync_copy(x_hbm.at[pl.ds(base, _RT)], x_v)

    def rowfn(r, carry):
        for c in range(_D // 16):
            sl = pl.ds(c * 16, 16)
            xx = x_v[r, sl]
            q_v[r, sl] = xx + (q_v[r, sl] - xx)
        return carry

    lax.fori_loop(0, _RT, rowfn, 0)

    zero16 = jnp.zeros((16,), jnp.float32)

    def zfn(i, carry):
        hist_v[pl.ds(i * 16, 16)] = zero16
        return carry

    lax.fori_loop(0, _KP // 16, zfn, 0)

    ones16 = jnp.ones((16,), jnp.float32)
    lanes = lax.iota(jnp.int32, 16)

    def hfn(t, carry):
        ivec = idx_v[t // 8, pl.ds((t % 8) * 16, 16)]
        for l in range(16):
            plsc.addupdate_scatter(hist_v, [ivec], ones16,
                                   mask=lanes == l)
        return carry

    lax.fori_loop(0, _NW, hfn, 0)

    pltpu.sync_copy(q_v, outq_hbm.at[pl.ds(base, _RT)])
    pltpu.sync_copy(hist_v, hist_hbm.at[wid])


@functools.cache
def _sc_gather_hist():
    mesh = plsc.VectorSubcoreMesh(core_axis_name="c", subcore_axis_name="s")
    return pl.kernel(
        _sc_body,
        mesh=mesh,
        out_type=(jax.ShapeDtypeStruct((_M, _D), jnp.float32),
                  jax.ShapeDtypeStruct((_NW, _KP), jnp.float32)),
        scratch_types=(pltpu.VMEM((4, 128), jnp.int32),
                       pltpu.VMEM((_RT, _D), jnp.float32),
                       pltpu.VMEM((_RT, _D), jnp.float32),
                       pltpu.VMEM((_KP,), jnp.float32),
                       pltpu.SemaphoreType.DMA),
        compiler_params=pltpu.CompilerParams(needs_layout_passes=False,
                                             use_tc_tiling_on_sc=False),
    )


def _vq1_call(flat, w):
    return pl.pallas_call(
        _vq1_body,
        grid=(_M // _RT,),
        in_specs=[pl.BlockSpec((_RT, _D), lambda i: (i, 0)),
                  pl.BlockSpec((_KP, _D), lambda i: (0, 0))],
        out_specs=[pl.BlockSpec((_RT, 1), lambda i: (i, 0)),
                   pl.BlockSpec((_RT, 1), lambda i: (i, 0))],
        out_shape=[jax.ShapeDtypeStruct((_M, 1), jnp.int32),
                   jax.ShapeDtypeStruct((_M, 1), jnp.float32)],
    )(flat, w)


def _head_call(hist16, mind, w1, b1, g1, bb1, w2, b2, g2, bb2, sw):
    return pl.pallas_call(
        _head_body,
        out_shape=[jax.ShapeDtypeStruct((_B, _D), jnp.float32),
                   jax.ShapeDtypeStruct((1, 1), jnp.float32),
                   jax.ShapeDtypeStruct((_B, 1), jnp.int32)],
    )(hist16, mind, w1, b1, g1, bb1, w2, b2, g2, bb2, sw)


def kernel(patch_embeddings, patch_embed_w, state_embed_w, W1, b1,
           ln1_g, ln1_b, W2, b2, ln2_g, ln2_b):
    flat = patch_embeddings.reshape(_M, _D)
    idx_col, mind_col = _vq1_call(flat, patch_embed_w)
    patch_indices = idx_col.reshape(_B, _N)
    idx_sc = idx_col.reshape(_NW, 4, 128)
    patch_q_flat, hist = _sc_gather_hist()(flat, patch_embed_w, idx_sc)
    patch_quantized = patch_q_flat.reshape(_B, _N, _D)
    sq, loss11, sidx = _head_call(
        hist.reshape(_B, 2 * _KP), mind_col.reshape(128, 128),
        W1, b1.reshape(1, _D), ln1_g.reshape(1, _D), ln1_b.reshape(1, _D),
        W2, b2.reshape(1, _D), ln2_g.reshape(1, _D), ln2_b.reshape(1, _D),
        state_embed_w)
    return (patch_quantized, sq, loss11.reshape(()), patch_indices,
            sidx.reshape(_B))


# speedup vs baseline: 1.2755x; 1.0766x over previous
import functools

import jax
import jax.numpy as jnp
from jax import lax
from jax.experimental import pallas as pl
from jax.experimental.pallas import tpu as pltpu
from jax.experimental.pallas import tpu_sc as plsc

_M = 16384
_D = 64
_KP = 8192
_KS = 1024
_B = 16
_N = 1024
_CC = 0.25
_RT = 512
_KT = 1024
_STRIP = 4096
_NW = 32


def _vq1_body(x_ref, w_ref, idx_ref, mind_ref):
    x = x_ref[...]
    q = x * x
    g = q[:, 0:8]
    for t in range(1, 8):
        g = g + q[:, 8 * t:8 * t + 8]
    h2 = g[:, 0:4] + g[:, 4:8]
    i2 = h2[:, 0:2] + h2[:, 2:4]
    x2 = i2[:, 0:1] + i2[:, 1:2]
    xb = (2.0 * x).astype(jnp.bfloat16).astype(jnp.float32)
    ones_row = jnp.ones((1, _D), jnp.float32)
    iota_k = lax.broadcasted_iota(jnp.int32, (_RT, _KT), 1)

    def sub_step(j, carry):
        sv, si = carry
        wk = w_ref[pl.ds(j * _KT, _KT), :]
        xw2 = lax.dot_general(xb, wk, (((1,), (1,)), ((), ())),
                              preferred_element_type=jnp.float32)
        w2 = lax.dot_general(ones_row, wk * wk, (((1,), (1,)), ((), ())),
                             preferred_element_type=jnp.float32)
        d = (x2 + w2) - xw2
        lm = jnp.min(d, axis=1, keepdims=True)
        li = jnp.min(jnp.where(d == lm, iota_k, _KT), axis=1, keepdims=True)
        pred = lm < sv
        return (jnp.where(pred, lm, sv),
                jnp.where(pred, li + j * _KT, si))

    def strip_step(c, carry):
        bv, bi, be = carry
        sv0 = jnp.full((_RT, 1), jnp.inf, jnp.float32)
        si0 = jnp.zeros((_RT, 1), jnp.int32)
        n_sub = _STRIP // _KT
        sv, si = lax.fori_loop(c * n_sub, (c + 1) * n_sub, sub_step,
                               (sv0, si0))
        pred = sv < bv
        bv = jnp.where(pred, sv, bv).astype(jnp.bfloat16).astype(jnp.float32)
        return (bv, jnp.where(pred, si, bi), jnp.where(pred, sv, be))

    bv0 = jnp.full((_RT, 1), jnp.inf, jnp.float32)
    bi0 = jnp.zeros((_RT, 1), jnp.int32)
    be0 = jnp.full((_RT, 1), jnp.inf, jnp.float32)
    _, bi, be = lax.fori_loop(0, _KP // _STRIP, strip_step, (bv0, bi0, be0))
    idx_ref[...] = bi
    mind_ref[...] = be


def _head_body(hist_ref, mind_ref, w1_ref, b1_ref, g1_ref, bb1_ref,
               w2_ref, b2_ref, g2_ref, bb2_ref, sw_ref,
               sq_ref, loss_ref, sidx_ref):
    hist = hist_ref[...]
    pd = (hist[:, :_KP] + hist[:, _KP:]) * (1.0 / _N)
    h = lax.dot_general(pd, w1_ref[...], (((1,), (0,)), ((), ())),
                        preferred_element_type=jnp.float32) + b1_ref[...]

    def ln(v, g, b):
        mu = jnp.mean(v, axis=-1, keepdims=True)
        var = jnp.mean((v - mu) ** 2, axis=-1, keepdims=True)
        return (v - mu) / jnp.sqrt(var + 1e-5) * g + b

    h = ln(h, g1_ref[...], bb1_ref[...])
    h = jnp.maximum(h, 0.0)
    h = lax.dot_general(h, w2_ref[...], (((1,), (0,)), ((), ())),
                        preferred_element_type=jnp.float32) + b2_ref[...]
    s = ln(h, g2_ref[...], bb2_ref[...])

    sw = sw_ref[...]
    s2 = jnp.sum(s * s, axis=1, keepdims=True)
    ones_row = jnp.ones((1, _D), jnp.float32)
    sw2 = lax.dot_general(ones_row, sw * sw, (((1,), (1,)), ((), ())),
                          preferred_element_type=jnp.float32)
    ssw = lax.dot_general(s, sw, (((1,), (1,)), ((), ())),
                          preferred_element_type=jnp.float32)
    d2 = (s2 + sw2) - 2.0 * ssw
    m2 = jnp.min(d2, axis=1, keepdims=True)
    iota_s = lax.broadcasted_iota(jnp.int32, (_B, _KS), 1)
    si = jnp.min(jnp.where(d2 == m2, iota_s, _KS), axis=1, keepdims=True)
    oh = (iota_s == si).astype(jnp.float32)
    q2 = lax.dot_general(oh, sw, (((1,), (0,)), ((), ())),
                         preferred_element_type=jnp.float32)

    diff = q2 - s
    msq = jnp.sum(diff * diff) * (1.0 / (_B * _D))
    state_loss = msq + _CC * msq
    pm = jnp.sum(mind_ref[...]) * (1.0 / (_M * _D))
    patch_loss = pm + _CC * pm
    total = patch_loss + state_loss

    sq_ref[...] = s + (q2 - s)
    loss_ref[...] = jnp.reshape(total, (1, 1))
    sidx_ref[...] = si


def _sc_body(x_hbm, w_hbm, idx_hbm, outq_hbm, hist_hbm,
             idx_v, q_v, x_v, hist_v, sem):
    wid = lax.axis_index("s") * 2 + lax.axis_index("c")
    base = wid * _RT
    pltpu.sync_copy(idx_hbm.at[wid], idx_v)
    for j in range(4):
        pltpu.async_copy(w_hbm.at[idx_v.at[j]],
                         q_v.at[pl.ds(j * 128, 128)], sem).wait()
    pltpu.sync_copy(x_hbm.at[pl.ds(base, _RT)], x_v)

    def rowfn(r, carry):
        for c in range(_D // 16):
            sl = pl.ds(c * 16, 16)
            xx = x_v[r, sl]
            q_v[r, sl] = xx + (q_v[r, sl] - xx)
        return carry

    lax.fori_loop(0, _RT, rowfn, 0)

    zero16 = jnp.zeros((16,), jnp.float32)

    def zfn(i, carry):
        hist_v[pl.ds(i * 16, 16)] = zero16
        return carry

    lax.fori_loop(0, _KP // 16, zfn, 0)

    ones16 = jnp.ones((16,), jnp.float32)
    lanes = lax.iota(jnp.int32, 16)

    def hfn(t, carry):
        ivec = idx_v[t // 8, pl.ds((t % 8) * 16, 16)]
        for l in range(16):
            plsc.addupdate_scatter(hist_v, [ivec], ones16,
                                   mask=lanes == l)
        return carry

    lax.fori_loop(0, _NW, hfn, 0)

    pltpu.sync_copy(q_v, outq_hbm.at[pl.ds(base, _RT)])
    pltpu.sync_copy(hist_v, hist_hbm.at[wid])


@functools.cache
def _sc_gather_hist():
    mesh = plsc.VectorSubcoreMesh(core_axis_name="c", subcore_axis_name="s")
    return pl.kernel(
        _sc_body,
        mesh=mesh,
        out_type=(jax.ShapeDtypeStruct((_M, _D), jnp.float32),
                  jax.ShapeDtypeStruct((_NW, _KP), jnp.float32)),
        scratch_types=(pltpu.VMEM((4, 128), jnp.int32),
                       pltpu.VMEM((_RT, _D), jnp.float32),
                       pltpu.VMEM((_RT, _D), jnp.float32),
                       pltpu.VMEM((_KP,), jnp.float32),
                       pltpu.SemaphoreType.DMA),
        compiler_params=pltpu.CompilerParams(needs_layout_passes=False,
                                             use_tc_tiling_on_sc=False),
    )


def _vq1_call(flat, w):
    return pl.pallas_call(
        _vq1_body,
        grid=(_M // _RT,),
        in_specs=[pl.BlockSpec((_RT, _D), lambda i: (i, 0)),
                  pl.BlockSpec((_KP, _D), lambda i: (0, 0))],
        out_specs=[pl.BlockSpec((_RT, 1), lambda i: (i, 0)),
                   pl.BlockSpec((_RT, 1), lambda i: (i, 0))],
        out_shape=[jax.ShapeDtypeStruct((_M, 1), jnp.int32),
                   jax.ShapeDtypeStruct((_M, 1), jnp.float32)],
    )(flat, w)


def _head_call(hist16, mind, w1, b1, g1, bb1, w2, b2, g2, bb2, sw):
    return pl.pallas_call(
        _head_body,
        out_shape=[jax.ShapeDtypeStruct((_B, _D), jnp.float32),
                   jax.ShapeDtypeStruct((1, 1), jnp.float32),
                   jax.ShapeDtypeStruct((_B, 1), jnp.int32)],
    )(hist16, mind, w1, b1, g1, bb1, w2, b2, g2, bb2, sw)


def kernel(patch_embeddings, patch_embed_w, state_embed_w, W1, b1,
           ln1_g, ln1_b, W2, b2, ln2_g, ln2_b):
    flat = patch_embeddings.reshape(_M, _D)
    idx_col, mind_col = _vq1_call(flat, patch_embed_w)
    patch_indices = idx_col.reshape(_B, _N)
    idx_sc = idx_col.reshape(_NW, 4, 128)
    patch_q_flat, hist = _sc_gather_hist()(flat, patch_embed_w, idx_sc)
    patch_quantized = patch_q_flat.reshape(_B, _N, _D)
    sq, loss11, sidx = _head_call(
        hist.reshape(_B, 2 * _KP), mind_col.reshape(128, 128),
        W1, b1.reshape(1, _D), ln1_g.reshape(1, _D), ln1_b.reshape(1, _D),
        W2, b2.reshape(1, _D), ln2_g.reshape(1, _D), ln2_b.reshape(1, _D),
        state_embed_w)
    return (patch_quantized, sq, loss11.reshape(()), patch_indices,
            sidx.reshape(_B))


# speedup vs baseline: 1.4453x; 1.1332x over previous
import functools

import jax
import jax.numpy as jnp
from jax import lax
from jax.experimental import pallas as pl
from jax.experimental.pallas import tpu as pltpu
from jax.experimental.pallas import tpu_sc as plsc

_M = 16384
_D = 64
_KP = 8192
_KS = 1024
_B = 16
_N = 1024
_CC = 0.25
_RT = 512
_KT = 2048
_STRIP = 4096
_NW = 32


def _vq1_body(x_ref, w_ref, idx_ref, mind_ref):
    x = x_ref[...]
    q = x * x
    g = q[:, 0:8]
    for t in range(1, 8):
        g = g + q[:, 8 * t:8 * t + 8]
    h2 = g[:, 0:4] + g[:, 4:8]
    i2 = h2[:, 0:2] + h2[:, 2:4]
    x2 = i2[:, 0:1] + i2[:, 1:2]
    xb = (2.0 * x).astype(jnp.bfloat16).astype(jnp.float32)
    ones_row = jnp.ones((1, _D), jnp.float32)
    iota_k = lax.broadcasted_iota(jnp.int32, (_RT, _KT), 1)

    def sub_step(j, carry):
        sv, si = carry
        wk = w_ref[pl.ds(j * _KT, _KT), :]
        xw2 = lax.dot_general(xb, wk, (((1,), (1,)), ((), ())),
                              preferred_element_type=jnp.float32)
        w2 = lax.dot_general(ones_row, wk * wk, (((1,), (1,)), ((), ())),
                             preferred_element_type=jnp.float32)
        d = (x2 + w2) - xw2
        lm = jnp.min(d, axis=1, keepdims=True)
        li = jnp.min(jnp.where(d == lm, iota_k, _KT), axis=1, keepdims=True)
        pred = lm < sv
        return (jnp.where(pred, lm, sv),
                jnp.where(pred, li + j * _KT, si))

    def strip_step(c, carry):
        bv, bi, be = carry
        sv0 = jnp.full((_RT, 1), jnp.inf, jnp.float32)
        si0 = jnp.zeros((_RT, 1), jnp.int32)
        n_sub = _STRIP // _KT
        sv, si = lax.fori_loop(c * n_sub, (c + 1) * n_sub, sub_step,
                               (sv0, si0))
        pred = sv < bv
        bv = jnp.where(pred, sv, bv).astype(jnp.bfloat16).astype(jnp.float32)
        return (bv, jnp.where(pred, si, bi), jnp.where(pred, sv, be))

    bv0 = jnp.full((_RT, 1), jnp.inf, jnp.float32)
    bi0 = jnp.zeros((_RT, 1), jnp.int32)
    be0 = jnp.full((_RT, 1), jnp.inf, jnp.float32)
    _, bi, be = lax.fori_loop(0, _KP // _STRIP, strip_step, (bv0, bi0, be0))
    idx_ref[...] = bi
    mind_ref[...] = be


def _head_body(hist_ref, mind_ref, w1_ref, b1_ref, g1_ref, bb1_ref,
               w2_ref, b2_ref, g2_ref, bb2_ref, sw_ref,
               sq_ref, loss_ref, sidx_ref):
    hist = hist_ref[...]
    pd = (hist[:, :_KP] + hist[:, _KP:]) * (1.0 / _N)
    h = lax.dot_general(pd, w1_ref[...], (((1,), (0,)), ((), ())),
                        preferred_element_type=jnp.float32) + b1_ref[...]

    def ln(v, g, b):
        mu = jnp.mean(v, axis=-1, keepdims=True)
        var = jnp.mean((v - mu) ** 2, axis=-1, keepdims=True)
        return (v - mu) / jnp.sqrt(var + 1e-5) * g + b

    h = ln(h, g1_ref[...], bb1_ref[...])
    h = jnp.maximum(h, 0.0)
    h = lax.dot_general(h, w2_ref[...], (((1,), (0,)), ((), ())),
                        preferred_element_type=jnp.float32) + b2_ref[...]
    s = ln(h, g2_ref[...], bb2_ref[...])

    sw = sw_ref[...]
    s2 = jnp.sum(s * s, axis=1, keepdims=True)
    ones_row = jnp.ones((1, _D), jnp.float32)
    sw2 = lax.dot_general(ones_row, sw * sw, (((1,), (1,)), ((), ())),
                          preferred_element_type=jnp.float32)
    ssw = lax.dot_general(s, sw, (((1,), (1,)), ((), ())),
                          preferred_element_type=jnp.float32)
    d2 = (s2 + sw2) - 2.0 * ssw
    m2 = jnp.min(d2, axis=1, keepdims=True)
    iota_s = lax.broadcasted_iota(jnp.int32, (_B, _KS), 1)
    si = jnp.min(jnp.where(d2 == m2, iota_s, _KS), axis=1, keepdims=True)
    oh = (iota_s == si).astype(jnp.float32)
    q2 = lax.dot_general(oh, sw, (((1,), (0,)), ((), ())),
                         preferred_element_type=jnp.float32)

    diff = q2 - s
    msq = jnp.sum(diff * diff) * (1.0 / (_B * _D))
    state_loss = msq + _CC * msq
    pm = jnp.sum(mind_ref[...]) * (1.0 / (_M * _D))
    patch_loss = pm + _CC * pm
    total = patch_loss + state_loss

    sq_ref[...] = s + (q2 - s)
    loss_ref[...] = jnp.reshape(total, (1, 1))
    sidx_ref[...] = si


def _sc_body(x_hbm, w_hbm, idx_hbm, outq_hbm, hist_hbm,
             idx_v, q_v, x_v, hist_v, sem):
    wid = lax.axis_index("s") * 2 + lax.axis_index("c")
    base = wid * _RT
    pltpu.sync_copy(idx_hbm.at[wid], idx_v)
    for j in range(4):
        pltpu.async_copy(w_hbm.at[idx_v.at[j]],
                         q_v.at[pl.ds(j * 128, 128)], sem).wait()
    pltpu.sync_copy(x_hbm.at[pl.ds(base, _RT)], x_v)

    def rowfn(r, carry):
        for c in range(_D // 16):
            sl = pl.ds(c * 16, 16)
            xx = x_v[r, sl]
            q_v[r, sl] = xx + (q_v[r, sl] - xx)
        return carry

    lax.fori_loop(0, _RT, rowfn, 0)

    zero16 = jnp.zeros((16,), jnp.float32)

    def zfn(i, carry):
        hist_v[pl.ds(i * 16, 16)] = zero16
        return carry

    lax.fori_loop(0, _KP // 16, zfn, 0)

    ones16 = jnp.ones((16,), jnp.float32)
    lanes = lax.iota(jnp.int32, 16)

    def hfn(t, carry):
        ivec = idx_v[t // 8, pl.ds((t % 8) * 16, 16)]
        for l in range(16):
            plsc.addupdate_scatter(hist_v, [ivec], ones16,
                                   mask=lanes == l)
        return carry

    lax.fori_loop(0, _NW, hfn, 0)

    pltpu.sync_copy(q_v, outq_hbm.at[pl.ds(base, _RT)])
    pltpu.sync_copy(hist_v, hist_hbm.at[wid])


@functools.cache
def _sc_gather_hist():
    mesh = plsc.VectorSubcoreMesh(core_axis_name="c", subcore_axis_name="s")
    return pl.kernel(
        _sc_body,
        mesh=mesh,
        out_type=(jax.ShapeDtypeStruct((_M, _D), jnp.float32),
                  jax.ShapeDtypeStruct((_NW, _KP), jnp.float32)),
        scratch_types=(pltpu.VMEM((4, 128), jnp.int32),
                       pltpu.VMEM((_RT, _D), jnp.float32),
                       pltpu.VMEM((_RT, _D), jnp.float32),
                       pltpu.VMEM((_KP,), jnp.float32),
                       pltpu.SemaphoreType.DMA),
        compiler_params=pltpu.CompilerParams(needs_layout_passes=False,
                                             use_tc_tiling_on_sc=False),
    )


def _vq1_call(flat, w):
    return pl.pallas_call(
        _vq1_body,
        grid=(_M // _RT,),
        in_specs=[pl.BlockSpec((_RT, _D), lambda i: (i, 0)),
                  pl.BlockSpec((_KP, _D), lambda i: (0, 0))],
        out_specs=[pl.BlockSpec((_RT, 1), lambda i: (i, 0)),
                   pl.BlockSpec((_RT, 1), lambda i: (i, 0))],
        out_shape=[jax.ShapeDtypeStruct((_M, 1), jnp.int32),
                   jax.ShapeDtypeStruct((_M, 1), jnp.float32)],
    )(flat, w)


def _head_call(hist16, mind, w1, b1, g1, bb1, w2, b2, g2, bb2, sw):
    return pl.pallas_call(
        _head_body,
        out_shape=[jax.ShapeDtypeStruct((_B, _D), jnp.float32),
                   jax.ShapeDtypeStruct((1, 1), jnp.float32),
                   jax.ShapeDtypeStruct((_B, 1), jnp.int32)],
    )(hist16, mind, w1, b1, g1, bb1, w2, b2, g2, bb2, sw)


def kernel(patch_embeddings, patch_embed_w, state_embed_w, W1, b1,
           ln1_g, ln1_b, W2, b2, ln2_g, ln2_b):
    flat = patch_embeddings.reshape(_M, _D)
    idx_col, mind_col = _vq1_call(flat, patch_embed_w)
    patch_indices = idx_col.reshape(_B, _N)
    idx_sc = idx_col.reshape(_NW, 4, 128)
    patch_q_flat, hist = _sc_gather_hist()(flat, patch_embed_w, idx_sc)
    patch_quantized = patch_q_flat.reshape(_B, _N, _D)
    sq, loss11, sidx = _head_call(
        hist.reshape(_B, 2 * _KP), mind_col.reshape(128, 128),
        W1, b1.reshape(1, _D), ln1_g.reshape(1, _D), ln1_b.reshape(1, _D),
        W2, b2.reshape(1, _D), ln2_g.reshape(1, _D), ln2_b.reshape(1, _D),
        state_embed_w)
    return (patch_quantized, sq, loss11.reshape(()), patch_indices,
            sidx.reshape(_B))


# speedup vs baseline: 1.5467x; 1.0701x over previous
import functools

import jax
import jax.numpy as jnp
from jax import lax
from jax.experimental import pallas as pl
from jax.experimental.pallas import tpu as pltpu
from jax.experimental.pallas import tpu_sc as plsc

_M = 16384
_D = 64
_KP = 8192
_KS = 1024
_B = 16
_N = 1024
_CC = 0.25
_RT = 512
_KT = 4096
_STRIP = 4096
_NW = 32


def _vq1_body(x_ref, w_ref, idx_ref, mind_ref):
    x = x_ref[...]
    q = x * x
    g = q[:, 0:8]
    for t in range(1, 8):
        g = g + q[:, 8 * t:8 * t + 8]
    h2 = g[:, 0:4] + g[:, 4:8]
    i2 = h2[:, 0:2] + h2[:, 2:4]
    x2 = i2[:, 0:1] + i2[:, 1:2]
    xb = (2.0 * x).astype(jnp.bfloat16).astype(jnp.float32)
    ones_row = jnp.ones((1, _D), jnp.float32)
    iota_k = lax.broadcasted_iota(jnp.int32, (_RT, _KT), 1)

    def sub_step(j, carry):
        sv, si = carry
        wk = w_ref[pl.ds(j * _KT, _KT), :]
        xw2 = lax.dot_general(xb, wk, (((1,), (1,)), ((), ())),
                              preferred_element_type=jnp.float32)
        w2 = lax.dot_general(ones_row, wk * wk, (((1,), (1,)), ((), ())),
                             preferred_element_type=jnp.float32)
        d = (x2 + w2) - xw2
        lm = jnp.min(d, axis=1, keepdims=True)
        li = jnp.min(jnp.where(d == lm, iota_k, _KT), axis=1, keepdims=True)
        pred = lm < sv
        return (jnp.where(pred, lm, sv),
                jnp.where(pred, li + j * _KT, si))

    def strip_step(c, carry):
        bv, bi, be = carry
        sv0 = jnp.full((_RT, 1), jnp.inf, jnp.float32)
        si0 = jnp.zeros((_RT, 1), jnp.int32)
        n_sub = _STRIP // _KT
        sv, si = lax.fori_loop(c * n_sub, (c + 1) * n_sub, sub_step,
                               (sv0, si0))
        pred = sv < bv
        bv = jnp.where(pred, sv, bv).astype(jnp.bfloat16).astype(jnp.float32)
        return (bv, jnp.where(pred, si, bi), jnp.where(pred, sv, be))

    bv0 = jnp.full((_RT, 1), jnp.inf, jnp.float32)
    bi0 = jnp.zeros((_RT, 1), jnp.int32)
    be0 = jnp.full((_RT, 1), jnp.inf, jnp.float32)
    _, bi, be = lax.fori_loop(0, _KP // _STRIP, strip_step, (bv0, bi0, be0))
    idx_ref[...] = bi
    mind_ref[...] = be


def _head_body(hist_ref, mind_ref, w1_ref, b1_ref, g1_ref, bb1_ref,
               w2_ref, b2_ref, g2_ref, bb2_ref, sw_ref,
               sq_ref, loss_ref, sidx_ref):
    hist = hist_ref[...]
    pd = (hist[:, :_KP] + hist[:, _KP:]) * (1.0 / _N)
    h = lax.dot_general(pd, w1_ref[...], (((1,), (0,)), ((), ())),
                        preferred_element_type=jnp.float32) + b1_ref[...]

    def ln(v, g, b):
        mu = jnp.mean(v, axis=-1, keepdims=True)
        var = jnp.mean((v - mu) ** 2, axis=-1, keepdims=True)
        return (v - mu) / jnp.sqrt(var + 1e-5) * g + b

    h = ln(h, g1_ref[...], bb1_ref[...])
    h = jnp.maximum(h, 0.0)
    h = lax.dot_general(h, w2_ref[...], (((1,), (0,)), ((), ())),
                        preferred_element_type=jnp.float32) + b2_ref[...]
    s = ln(h, g2_ref[...], bb2_ref[...])

    sw = sw_ref[...]
    s2 = jnp.sum(s * s, axis=1, keepdims=True)
    ones_row = jnp.ones((1, _D), jnp.float32)
    sw2 = lax.dot_general(ones_row, sw * sw, (((1,), (1,)), ((), ())),
                          preferred_element_type=jnp.float32)
    ssw = lax.dot_general(s, sw, (((1,), (1,)), ((), ())),
                          preferred_element_type=jnp.float32)
    d2 = (s2 + sw2) - 2.0 * ssw
    m2 = jnp.min(d2, axis=1, keepdims=True)
    iota_s = lax.broadcasted_iota(jnp.int32, (_B, _KS), 1)
    si = jnp.min(jnp.where(d2 == m2, iota_s, _KS), axis=1, keepdims=True)
    oh = (iota_s == si).astype(jnp.float32)
    q2 = lax.dot_general(oh, sw, (((1,), (0,)), ((), ())),
                         preferred_element_type=jnp.float32)

    diff = q2 - s
    msq = jnp.sum(diff * diff) * (1.0 / (_B * _D))
    state_loss = msq + _CC * msq
    pm = jnp.sum(mind_ref[...]) * (1.0 / (_M * _D))
    patch_loss = pm + _CC * pm
    total = patch_loss + state_loss

    sq_ref[...] = s + (q2 - s)
    loss_ref[...] = jnp.reshape(total, (1, 1))
    sidx_ref[...] = si


def _sc_body(x_hbm, w_hbm, idx_hbm, outq_hbm, hist_hbm,
             idx_v, q_v, x_v, hist_v, sem):
    wid = lax.axis_index("s") * 2 + lax.axis_index("c")
    base = wid * _RT
    pltpu.sync_copy(idx_hbm.at[wid], idx_v)
    for j in range(4):
        pltpu.async_copy(w_hbm.at[idx_v.at[j]],
                         q_v.at[pl.ds(j * 128, 128)], sem).wait()
    pltpu.sync_copy(x_hbm.at[pl.ds(base, _RT)], x_v)

    def rowfn(r, carry):
        for c in range(_D // 16):
            sl = pl.ds(c * 16, 16)
            xx = x_v[r, sl]
            q_v[r, sl] = xx + (q_v[r, sl] - xx)
        return carry

    lax.fori_loop(0, _RT, rowfn, 0)

    zero16 = jnp.zeros((16,), jnp.float32)

    def zfn(i, carry):
        hist_v[pl.ds(i * 16, 16)] = zero16
        return carry

    lax.fori_loop(0, _KP // 16, zfn, 0)

    ones16 = jnp.ones((16,), jnp.float32)
    lanes = lax.iota(jnp.int32, 16)

    def hfn(t, carry):
        ivec = idx_v[t // 8, pl.ds((t % 8) * 16, 16)]
        for l in range(16):
            plsc.addupdate_scatter(hist_v, [ivec], ones16,
                                   mask=lanes == l)
        return carry

    lax.fori_loop(0, _NW, hfn, 0)

    pltpu.sync_copy(q_v, outq_hbm.at[pl.ds(base, _RT)])
    pltpu.sync_copy(hist_v, hist_hbm.at[wid])


@functools.cache
def _sc_gather_hist():
    mesh = plsc.VectorSubcoreMesh(core_axis_name="c", subcore_axis_name="s")
    return pl.kernel(
        _sc_body,
        mesh=mesh,
        out_type=(jax.ShapeDtypeStruct((_M, _D), jnp.float32),
                  jax.ShapeDtypeStruct((_NW, _KP), jnp.float32)),
        scratch_types=(pltpu.VMEM((4, 128), jnp.int32),
                       pltpu.VMEM((_RT, _D), jnp.float32),
                       pltpu.VMEM((_RT, _D), jnp.float32),
                       pltpu.VMEM((_KP,), jnp.float32),
                       pltpu.SemaphoreType.DMA),
        compiler_params=pltpu.CompilerParams(needs_layout_passes=False,
                                             use_tc_tiling_on_sc=False),
    )


def _vq1_call(flat, w):
    return pl.pallas_call(
        _vq1_body,
        grid=(_M // _RT,),
        in_specs=[pl.BlockSpec((_RT, _D), lambda i: (i, 0)),
                  pl.BlockSpec((_KP, _D), lambda i: (0, 0))],
        out_specs=[pl.BlockSpec((_RT, 1), lambda i: (i, 0)),
                   pl.BlockSpec((_RT, 1), lambda i: (i, 0))],
        out_shape=[jax.ShapeDtypeStruct((_M, 1), jnp.int32),
                   jax.ShapeDtypeStruct((_M, 1), jnp.float32)],
    )(flat, w)


def _head_call(hist16, mind, w1, b1, g1, bb1, w2, b2, g2, bb2, sw):
    return pl.pallas_call(
        _head_body,
        out_shape=[jax.ShapeDtypeStruct((_B, _D), jnp.float32),
                   jax.ShapeDtypeStruct((1, 1), jnp.float32),
                   jax.ShapeDtypeStruct((_B, 1), jnp.int32)],
    )(hist16, mind, w1, b1, g1, bb1, w2, b2, g2, bb2, sw)


def kernel(patch_embeddings, patch_embed_w, state_embed_w, W1, b1,
           ln1_g, ln1_b, W2, b2, ln2_g, ln2_b):
    flat = patch_embeddings.reshape(_M, _D)
    idx_col, mind_col = _vq1_call(flat, patch_embed_w)
    patch_indices = idx_col.reshape(_B, _N)
    idx_sc = idx_col.reshape(_NW, 4, 128)
    patch_q_flat, hist = _sc_gather_hist()(flat, patch_embed_w, idx_sc)
    patch_quantized = patch_q_flat.reshape(_B, _N, _D)
    sq, loss11, sidx = _head_call(
        hist.reshape(_B, 2 * _KP), mind_col.reshape(128, 128),
        W1, b1.reshape(1, _D), ln1_g.reshape(1, _D), ln1_b.reshape(1, _D),
        W2, b2.reshape(1, _D), ln2_g.reshape(1, _D), ln2_b.reshape(1, _D),
        state_embed_w)
    return (patch_quantized, sq, loss11.reshape(()), patch_indices,
            sidx.reshape(_B))
